# Initial kernel scaffold; baseline (speedup 1.0000x reference)
import functools

import jax
import jax.numpy as jnp
from jax import lax
from jax.experimental import pallas as pl
from jax.experimental.pallas import tpu as pltpu
from jax.experimental.pallas import tpu_sc as plsc

_N_IN = 128
_N_OUT = 256
_E = 160000
_NC, _NS, _L = 2, 16, 16
_NW = _NC * _NS
_CHUNK = 128
_ROWS = 40
_EPT = _ROWS * _CHUNK
_E_PAD = _NW * _EPT
_ACC = _N_OUT * _N_IN
_TRASH = _ACC
_ACC_PAD = _ACC + 8
_SLICE = _ACC // _NS


def _sc_build_w(src_hbm, dst_hbm, w_hbm, out_hbm,
                src_v, idx_v, w_v, zero_v, acc_sh, sem0, sem1, sem2):
    c = lax.axis_index("c")
    s = lax.axis_index("s")
    gwid = c * _NS + s

    cp0 = pltpu.async_copy(src_hbm.at[gwid], src_v, sem0)
    cp1 = pltpu.async_copy(dst_hbm.at[gwid], idx_v, sem1)
    cp2 = pltpu.async_copy(w_hbm.at[gwid], w_v, sem2)

    def zero_body(i, carry):
        zero_v[pl.ds(i * _L, _L)] = jnp.zeros((_L,), jnp.float32)
        return carry
    lax.fori_loop(0, _SLICE // _L, zero_body, 0)
    pltpu.sync_copy(zero_v, acc_sh.at[pl.ds(s * _SLICE, _SLICE)])

    cp0.wait()
    cp1.wait()

    def idx_body(j, carry):
        for k in range(_CHUNK // _L):
            sl = pl.ds(k * _L, _L)
            sv = src_v[j, sl]
            dv = idx_v[j, sl]
            valid = (sv < _N_IN) & (dv < _N_OUT)
            idx_v[j, sl] = jnp.where(valid, dv * _N_IN + sv, _TRASH)
        return carry
    lax.fori_loop(0, _ROWS, idx_body, 0)

    cp2.wait()
    plsc.subcore_barrier()
    pltpu.sync_copy(w_v, acc_sh.at[idx_v], add=True)
    plsc.subcore_barrier()
    pltpu.sync_copy(acc_sh.at[pl.ds(s * _SLICE, _SLICE)], out_hbm.at[c, s])


_sc_kernel = functools.partial(
    pl.kernel,
    out_type=jax.ShapeDtypeStruct((_NC, _NS, _SLICE), jnp.float32),
    mesh=plsc.VectorSubcoreMesh(core_axis_name="c", subcore_axis_name="s",
                                num_cores=_NC, num_subcores=_NS),
    scratch_types=[
        pltpu.VMEM((_ROWS, _CHUNK), jnp.int32),
        pltpu.VMEM((_ROWS, _CHUNK), jnp.int32),
        pltpu.VMEM((_ROWS, _CHUNK), jnp.float32),
        pltpu.VMEM((_SLICE,), jnp.float32),
        pltpu.VMEM_SHARED((_ACC_PAD,), jnp.float32),
        pltpu.SemaphoreType.DMA,
        pltpu.SemaphoreType.DMA,
        pltpu.SemaphoreType.DMA,
    ],
)(_sc_build_w)


def _tc_body(x_ref, wp_ref, b_ref, r_ref, o_ref):
    w = wp_ref[0] + wp_ref[1]
    agg = lax.dot_general(x_ref[...], w, (((1,), (1,)), ((), ())),
                          preferred_element_type=jnp.float32,
                          precision=lax.Precision.HIGHEST)
    o_ref[...] = jnp.tanh(b_ref[...] + r_ref[...] * agg)


def kernel(inputs, weights, biases, responses, edge_src, edge_dst):
    batch = inputs.shape[0]
    pad = _E_PAD - _E
    src_p = jnp.concatenate(
        [edge_src, jnp.zeros((pad,), jnp.int32)]).reshape(_NW, _ROWS, _CHUNK)
    dst_p = jnp.concatenate(
        [edge_dst, jnp.full((pad,), _N_OUT, jnp.int32)]).reshape(_NW, _ROWS, _CHUNK)
    w_p = jnp.concatenate(
        [weights, jnp.zeros((pad,), jnp.float32)]).reshape(_NW, _ROWS, _CHUNK)

    wp = _sc_kernel(src_p, dst_p, w_p).reshape(_NC, _N_OUT, _N_IN)

    out = pl.pallas_call(
        _tc_body,
        out_shape=jax.ShapeDtypeStruct((batch, _N_OUT), jnp.float32),
    )(inputs, wp, biases[:_N_OUT].reshape(1, _N_OUT),
      responses[:_N_OUT].reshape(1, _N_OUT))
    return out


# speedup vs baseline: 29.4546x; 29.4546x over previous
import functools

import jax
import jax.numpy as jnp
from jax import lax
from jax.experimental import pallas as pl
from jax.experimental.pallas import tpu as pltpu
from jax.experimental.pallas import tpu_sc as plsc

_N_IN = 128
_N_OUT = 256
_E = 160000
_NC, _NS, _L = 2, 16, 16
_NW = _NC * _NS
_CHUNK = 128
_ROWS = 40
_EPT = _ROWS * _CHUNK
_E_PAD = _NW * _EPT
_ACC = _N_OUT * _N_IN


def _sc_build_w(src_hbm, dst_hbm, w_hbm, out_hbm,
                src_v, dst_v, w_v, acc_v, sem0, sem1, sem2):
    c = lax.axis_index("c")
    s = lax.axis_index("s")
    gwid = c * _NS + s

    cp0 = pltpu.async_copy(src_hbm.at[gwid], src_v, sem0)
    cp1 = pltpu.async_copy(dst_hbm.at[gwid], dst_v, sem1)
    cp2 = pltpu.async_copy(w_hbm.at[gwid], w_v, sem2)

    def zero_body(i, carry):
        acc_v[pl.ds(i * _L, _L)] = jnp.zeros((_L,), jnp.float32)
        return carry
    lax.fori_loop(0, _ACC // _L, zero_body, 0)

    cp0.wait()
    cp1.wait()
    cp2.wait()

    def edge_body(j, carry):
        for k in range(_CHUNK // _L):
            sl = pl.ds(k * _L, _L)
            sv = src_v[j, sl]
            dv = dst_v[j, sl]
            wv = w_v[j, sl]
            valid = (sv < _N_IN) & (dv < _N_OUT)
            idx = dv * _N_IN + sv
            plsc.addupdate_scatter(acc_v, [idx], wv, mask=valid)
        return carry
    lax.fori_loop(0, _ROWS, edge_body, 0)

    pltpu.sync_copy(acc_v, out_hbm.at[gwid])


_sc_kernel = functools.partial(
    pl.kernel,
    out_type=jax.ShapeDtypeStruct((_NW, _ACC), jnp.float32),
    mesh=plsc.VectorSubcoreMesh(core_axis_name="c", subcore_axis_name="s",
                                num_cores=_NC, num_subcores=_NS),
    compiler_params=pltpu.CompilerParams(needs_layout_passes=False),
    scratch_types=[
        pltpu.VMEM((_ROWS, _CHUNK), jnp.int32),
        pltpu.VMEM((_ROWS, _CHUNK), jnp.int32),
        pltpu.VMEM((_ROWS, _CHUNK), jnp.float32),
        pltpu.VMEM((_ACC,), jnp.float32),
        pltpu.SemaphoreType.DMA,
        pltpu.SemaphoreType.DMA,
        pltpu.SemaphoreType.DMA,
    ],
)(_sc_build_w)


def _tc_body(x_ref, wp_ref, b_ref, r_ref, o_ref):
    w = jnp.sum(wp_ref[...], axis=0)
    agg = lax.dot_general(x_ref[...], w, (((1,), (1,)), ((), ())),
                          preferred_element_type=jnp.float32,
                          precision=lax.Precision.HIGHEST)
    o_ref[...] = jnp.tanh(b_ref[...] + r_ref[...] * agg)


def kernel(inputs, weights, biases, responses, edge_src, edge_dst):
    batch = inputs.shape[0]
    pad = _E_PAD - _E
    src_p = jnp.concatenate(
        [edge_src, jnp.zeros((pad,), jnp.int32)]).reshape(_NW, _ROWS, _CHUNK)
    dst_p = jnp.concatenate(
        [edge_dst, jnp.full((pad,), _N_OUT, jnp.int32)]).reshape(_NW, _ROWS, _CHUNK)
    w_p = jnp.concatenate(
        [weights, jnp.zeros((pad,), jnp.float32)]).reshape(_NW, _ROWS, _CHUNK)

    wp = _sc_kernel(src_p, dst_p, w_p).reshape(_NW, _N_OUT, _N_IN)

    out = pl.pallas_call(
        _tc_body,
        out_shape=jax.ShapeDtypeStruct((batch, _N_OUT), jnp.float32),
    )(inputs, wp, biases[:_N_OUT].reshape(1, _N_OUT),
      responses[:_N_OUT].reshape(1, _N_OUT))
    return out
